# Initial kernel scaffold; baseline (speedup 1.0000x reference)
import jax
import jax.numpy as jnp
from jax.experimental import pallas as pl


def kernel(x, edge_index, Wl1, Wr1, att1, b1, Wlin1, blin1, Wl2, Wr2, att2, b2, Wlin2, blin2, Wlo, Wro, atto, bo, Wlino, blino):
    raise NotImplementedError("write your pallas kernel here")


# speedup vs baseline: 9.7018x; 9.7018x over previous
import functools

import jax
import jax.numpy as jnp
from jax import lax
from jax.experimental import pallas as pl
from jax.experimental.pallas import tpu as pltpu
from jax.experimental.pallas import tpu_sc as plsc

N = 10000
E = 320000
NC, NS, L = 2, 16, 16
NW = NC * NS
EW = E // NW
B = 80
NCHUNK = EW // B
RPT = N // NS


@functools.lru_cache(maxsize=None)
def _edge_phase(D):
    KD = D // L
    mesh = plsc.VectorSubcoreMesh(core_axis_name="c", subcore_axis_name="s",
                                  num_cores=NC, num_subcores=NS)

    @functools.partial(
        pl.kernel,
        mesh=mesh,
        compiler_params=pltpu.CompilerParams(use_tc_tiling_on_sc=False,
                                             needs_layout_passes=False),
        out_type=(jax.ShapeDtypeStruct((NC, N, D), jnp.float32),
                  jax.ShapeDtypeStruct((NC, N, 16), jnp.float32)),
        scratch_types=[
            pltpu.VMEM((B,), jnp.int32),
            pltpu.VMEM((B,), jnp.int32),
            pltpu.VMEM((B, D), jnp.float32),
            pltpu.VMEM((B, D), jnp.float32),
            pltpu.VMEM((B, 16), jnp.float32),
            pltpu.VMEM((D,), jnp.float32),
            pltpu.VMEM((RPT // 5, D), jnp.float32),
            pltpu.VMEM((RPT // 5, 16), jnp.float32),
            pltpu.VMEM_SHARED((N, D), jnp.float32),
            pltpu.VMEM_SHARED((N, 16), jnp.float32),
        ],
    )
    def edge_kernel(xl_hbm, xr_hbm, src_hbm, dst_hbm, att_hbm,
                    num_out, den_out,
                    src_v, dst_v, xl_v, xr_v, w_v, att_v, znum, zden,
                    num_sh, den_sh):
        cid = lax.axis_index("c")
        sid = lax.axis_index("s")
        wid = sid * NC + cid
        zero16 = jnp.zeros((L,), jnp.float32)

        def zinit(r, _):
            for k in range(KD):
                znum[r, pl.ds(k * L, L)] = zero16
            zden[r, :] = zero16
            return 0
        lax.fori_loop(0, RPT // 5, zinit, 0)
        row0 = sid * RPT
        for j in range(5):
            pltpu.sync_copy(znum, num_sh.at[pl.ds(row0 + j * (RPT // 5),
                                                  RPT // 5), :])
            pltpu.sync_copy(zden, den_sh.at[pl.ds(row0 + j * (RPT // 5),
                                                  RPT // 5), :])
        pltpu.sync_copy(att_hbm, att_v)
        plsc.subcore_barrier()

        att_k = [att_v[pl.ds(k * L, L)] for k in range(KD)]
        lane0 = (lax.iota(jnp.int32, L) == 0).astype(jnp.float32)
        base = wid * EW

        def do_chunk(c, _):
            off = base + c * B
            pltpu.sync_copy(src_hbm.at[pl.ds(off, B)], src_v)
            pltpu.sync_copy(dst_hbm.at[pl.ds(off, B)], dst_v)
            pltpu.sync_copy(xl_hbm.at[src_v], xl_v)
            pltpu.sync_copy(xr_hbm.at[dst_v], xr_v)

            def do_edge(e, _):
                xls = []
                acc = None
                for k in range(KD):
                    a = xl_v[e, pl.ds(k * L, L)]
                    b = xr_v[e, pl.ds(k * L, L)]
                    z = a + b
                    z = lax.max(z, z * 0.2)
                    t = z * att_k[k]
                    acc = t if acc is None else acc + t
                    xls.append(a)
                logit = jnp.sum(acc)
                wv = jnp.exp(jnp.full((L,), logit, jnp.float32))
                w_v[e, :] = wv * lane0
                for k in range(KD):
                    xl_v[e, pl.ds(k * L, L)] = xls[k] * wv
                return 0
            lax.fori_loop(0, B, do_edge, 0)

            pltpu.sync_copy(xl_v, num_sh.at[dst_v], add=True)
            pltpu.sync_copy(w_v, den_sh.at[dst_v], add=True)
            return 0
        lax.fori_loop(0, NCHUNK, do_chunk, 0)

        plsc.subcore_barrier()
        pltpu.sync_copy(num_sh.at[pl.ds(row0, RPT), :],
                        num_out.at[cid, pl.ds(row0, RPT), :])
        pltpu.sync_copy(den_sh.at[pl.ds(row0, RPT), :],
                        den_out.at[cid, pl.ds(row0, RPT), :])

    return edge_kernel


def _matmul(x, w):
    R = 1000
    K, M = w.shape

    def body(x_ref, w_ref, o_ref):
        o_ref[...] = jnp.dot(x_ref[...], w_ref[...],
                             preferred_element_type=jnp.float32)

    return pl.pallas_call(
        body,
        grid=(N // R,),
        in_specs=[pl.BlockSpec((R, K), lambda i: (i, 0)),
                  pl.BlockSpec((K, M), lambda i: (0, 0))],
        out_specs=pl.BlockSpec((R, M), lambda i: (i, 0)),
        out_shape=jax.ShapeDtypeStruct((N, M), jnp.float32),
    )(x, w)


def _combine_matmul(num, den, skip, bias, w):
    R = 1000
    D = num.shape[2]
    M = w.shape[1]

    def body(n_ref, d_ref, s_ref, b_ref, w_ref, o_ref):
        ns = n_ref[0] + n_ref[1]
        dsum = d_ref[0, :, 0:1] + d_ref[1, :, 0:1]
        h = ns / (dsum + 1e-16) + s_ref[...] + b_ref[...]
        h = jnp.maximum(h, 0.0)
        o_ref[...] = jnp.dot(h, w_ref[...],
                             preferred_element_type=jnp.float32)

    return pl.pallas_call(
        body,
        grid=(N // R,),
        in_specs=[pl.BlockSpec((NC, R, D), lambda i: (0, i, 0)),
                  pl.BlockSpec((NC, R, 16), lambda i: (0, i, 0)),
                  pl.BlockSpec((R, D), lambda i: (i, 0)),
                  pl.BlockSpec((1, D), lambda i: (0, 0)),
                  pl.BlockSpec((D, M), lambda i: (0, 0))],
        out_specs=pl.BlockSpec((R, M), lambda i: (i, 0)),
        out_shape=jax.ShapeDtypeStruct((N, M), jnp.float32),
    )(num, den, skip, bias, w)


def _final(num, den, skip, bias):
    R = 1000

    def body(n_ref, d_ref, s_ref, b_ref, o_ref):
        ns = n_ref[0] + n_ref[1]
        dsum = d_ref[0] + d_ref[1]
        o_ref[...] = ns / (dsum + 1e-16) + s_ref[...] + b_ref[...]

    return pl.pallas_call(
        body,
        grid=(N // R,),
        in_specs=[pl.BlockSpec((NC, R, 16), lambda i: (0, i, 0)),
                  pl.BlockSpec((NC, R, 16), lambda i: (0, i, 0)),
                  pl.BlockSpec((R, 16), lambda i: (i, 0)),
                  pl.BlockSpec((1, 16), lambda i: (0, 0))],
        out_specs=pl.BlockSpec((R, 16), lambda i: (i, 0)),
        out_shape=jax.ShapeDtypeStruct((N, 16), jnp.float32),
    )(num, den, skip, bias)


def kernel(x, edge_index, Wl1, Wr1, att1, b1, Wlin1, blin1,
           Wl2, Wr2, att2, b2, Wlin2, blin2,
           Wlo, Wro, atto, bo, Wlino, blino):
    src = edge_index[0]
    dst = edge_index[1]

    X1 = _matmul(x, jnp.concatenate([Wl1, Wr1, Wlin1], axis=1))
    num1, den1 = _edge_phase(128)(X1[:, :128], X1[:, 128:256], src, dst, att1)

    X2 = _combine_matmul(num1, den1, X1[:, 256:],
                         (b1 + blin1)[None, :],
                         jnp.concatenate([Wl2, Wr2, Wlin2], axis=1))
    num2, den2 = _edge_phase(128)(X2[:, :128], X2[:, 128:256], src, dst, att2)

    z = jnp.zeros((128, 15), jnp.float32)
    W3 = jnp.concatenate([Wlo, z, Wro, z, Wlino, z], axis=1)
    X3 = _combine_matmul(num2, den2, X2[:, 256:],
                         (b2 + blin2)[None, :], W3)

    att3 = jnp.concatenate([atto, jnp.zeros((15,), jnp.float32)])
    num3, den3 = _edge_phase(16)(X3[:, :16], X3[:, 16:32], src, dst, att3)

    bc = jnp.broadcast_to((bo + blino)[None, :], (1, 16))
    out16 = _final(num3, den3, X3[:, 32:48], bc)
    return out16[:, 0:1]


# speedup vs baseline: 17.6913x; 1.8235x over previous
import functools

import jax
import jax.numpy as jnp
from jax import lax
from jax.experimental import pallas as pl
from jax.experimental.pallas import tpu as pltpu
from jax.experimental.pallas import tpu_sc as plsc

N = 10000
E = 320000
NC, NS, L = 2, 16, 16
NW = NC * NS
EW = E // NW
B = 40
NCHUNK = EW // B
RPT = N // NS


@functools.lru_cache(maxsize=None)
def _edge_phase(D):
    KD = D // L
    DM = D + 16
    mesh = plsc.VectorSubcoreMesh(core_axis_name="c", subcore_axis_name="s",
                                  num_cores=NC, num_subcores=NS)

    @functools.partial(
        pl.kernel,
        mesh=mesh,
        compiler_params=pltpu.CompilerParams(use_tc_tiling_on_sc=False,
                                             needs_layout_passes=False),
        out_type=jax.ShapeDtypeStruct((NC, N, DM), jnp.float32),
        scratch_types=[
            pltpu.VMEM((4, B), jnp.int32),
            pltpu.VMEM((4, B), jnp.int32),
            pltpu.VMEM((B, D), jnp.float32),
            pltpu.VMEM((B, D), jnp.float32),
            pltpu.VMEM((B, D), jnp.float32),
            pltpu.VMEM((B, D), jnp.float32),
            pltpu.VMEM((B, DM), jnp.float32),
            pltpu.VMEM((B, DM), jnp.float32),
            pltpu.VMEM((D,), jnp.float32),
            pltpu.VMEM_SHARED((N, DM), jnp.float32),
            pltpu.SemaphoreType.DMA,
            pltpu.SemaphoreType.DMA,
            pltpu.SemaphoreType.DMA,
            pltpu.SemaphoreType.DMA,
            pltpu.SemaphoreType.DMA,
            pltpu.SemaphoreType.DMA,
            pltpu.SemaphoreType.DMA,
            pltpu.SemaphoreType.DMA,
        ],
    )
    def edge_kernel(xl_hbm, xr_hbm, src_hbm, dst_hbm, att_hbm, zer_hbm,
                    acc_out,
                    src_i, dst_i, xl0, xl1, xr0, xr1, m0, m1, att_v,
                    acc_sh, i0, i1, i2, i3, g0, g1, s0, s1):
        cid = lax.axis_index("c")
        sid = lax.axis_index("s")
        wid = sid * NC + cid
        xl_r, xr_r, msg = [xl0, xl1], [xr0, xr1], [m0, m1]
        isem, gsem, ssem = [i0, i1, i2, i3], [g0, g1], [s0, s1]

        row0 = sid * RPT
        pltpu.sync_copy(zer_hbm, acc_sh.at[pl.ds(row0, RPT), :])
        pltpu.sync_copy(att_hbm, att_v)
        plsc.subcore_barrier()

        att_k = [att_v[pl.ds(k * L, L)] for k in range(KD)]
        lane0 = (lax.iota(jnp.int32, L) == 0).astype(jnp.float32)
        base = wid * EW
        last = NCHUNK - 1

        def idx_fetch(chunk, slot, sem):
            off = base + chunk * B
            pltpu.make_async_copy(src_hbm.at[pl.ds(off, B)],
                                  src_i.at[slot], sem).start()
            pltpu.make_async_copy(dst_hbm.at[pl.ds(off, B)],
                                  dst_i.at[slot], sem).start()

        def idx_wait(slot, sem):
            pltpu.make_async_copy(src_hbm.at[pl.ds(base, B)],
                                  src_i.at[slot], sem).wait()
            pltpu.make_async_copy(dst_hbm.at[pl.ds(base, B)],
                                  dst_i.at[slot], sem).wait()

        def gather_start(slot, p):
            pltpu.make_async_copy(xl_hbm.at[src_i.at[slot]],
                                  xl_r[p], gsem[p]).start()
            pltpu.make_async_copy(xr_hbm.at[dst_i.at[slot]],
                                  xr_r[p], gsem[p]).start()

        def gather_wait(slot, p):
            pltpu.make_async_copy(xl_hbm.at[src_i.at[slot]],
                                  xl_r[p], gsem[p]).wait()
            pltpu.make_async_copy(xr_hbm.at[dst_i.at[slot]],
                                  xr_r[p], gsem[p]).wait()

        def scatter_start(slot, p):
            pltpu.make_async_copy(msg[p], acc_sh.at[dst_i.at[slot]],
                                  ssem[p]).start(add=True)

        def scatter_wait(slot, p):
            pltpu.make_async_copy(msg[p], acc_sh.at[dst_i.at[slot]],
                                  ssem[p]).wait()

        def compute(p):
            xlb, xrb, mb = xl_r[p], xr_r[p], msg[p]

            def one_edge(e):
                xls = []
                acc = None
                for k in range(KD):
                    a = xlb[e, pl.ds(k * L, L)]
                    b = xrb[e, pl.ds(k * L, L)]
                    z = a + b
                    z = lax.max(z, z * 0.2)
                    t = z * att_k[k]
                    acc = t if acc is None else acc + t
                    xls.append(a)
                wv = jnp.exp(jnp.full((L,), jnp.sum(acc), jnp.float32))
                mb[e, pl.ds(D, L)] = wv * lane0
                for k in range(KD):
                    mb[e, pl.ds(k * L, L)] = xls[k] * wv
                return 0

            def pair(j, _):
                one_edge(2 * j)
                one_edge(2 * j + 1)
                return 0
            lax.fori_loop(0, B // 2, pair, 0)

        def process(c, s_idx):
            p = s_idx % 2
            s_nxt, s_pre = (s_idx + 1) % 4, (s_idx + 2) % 4
            p_nxt = (p + 1) % 2
            gather_wait(s_idx, p)

            @pl.when(c >= 2)
            def _():
                scatter_wait(s_pre, p)
            idx_fetch(jnp.minimum(c + 2, last), s_pre, isem[s_pre])
            idx_wait(s_nxt, isem[s_nxt])
            gather_start(s_nxt, p_nxt)
            compute(p)
            scatter_start(s_idx, p)

        pltpu.sync_copy(src_hbm.at[pl.ds(base, B)], src_i.at[0])
        pltpu.sync_copy(dst_hbm.at[pl.ds(base, B)], dst_i.at[0])
        idx_fetch(1, 1, isem[1])
        gather_start(0, 0)

        def quad(i, _):
            for c4 in range(4):
                process(4 * i + c4, c4)
            return 0
        lax.fori_loop(0, NCHUNK // 4, quad, 0)
        for c in range(4 * (NCHUNK // 4), NCHUNK):
            process(c, c % 4)

        idx_wait((last + 2) % 4, isem[(last + 2) % 4])
        gather_wait((last + 1) % 4, (last + 1) % 2)
        scatter_wait((last - 1) % 4, (last - 1) % 2)
        scatter_wait(last % 4, last % 2)

        plsc.subcore_barrier()
        pltpu.sync_copy(acc_sh.at[pl.ds(row0, RPT), :],
                        acc_out.at[cid, pl.ds(row0, RPT), :])

    return edge_kernel


def _matmul(x, w):
    R = 1000
    K, M = w.shape

    def body(x_ref, w_ref, o_ref):
        o_ref[...] = jnp.dot(x_ref[...], w_ref[...],
                             preferred_element_type=jnp.float32)

    return pl.pallas_call(
        body,
        grid=(N // R,),
        in_specs=[pl.BlockSpec((R, K), lambda i: (i, 0)),
                  pl.BlockSpec((K, M), lambda i: (0, 0))],
        out_specs=pl.BlockSpec((R, M), lambda i: (i, 0)),
        out_shape=jax.ShapeDtypeStruct((N, M), jnp.float32),
    )(x, w)


def _combine_matmul(acc, skip, bias, w):
    R = 1000
    D = acc.shape[2] - 16
    M = w.shape[1]

    def body(a_ref, s_ref, b_ref, w_ref, o_ref):
        raw = a_ref[0] + a_ref[1]
        ns = raw[:, :D]
        den = raw[:, D:D + 1]
        h = ns / (den + 1e-16) + s_ref[...] + b_ref[...]
        h = jnp.maximum(h, 0.0)
        o_ref[...] = jnp.dot(h, w_ref[...],
                             preferred_element_type=jnp.float32)

    return pl.pallas_call(
        body,
        grid=(N // R,),
        in_specs=[pl.BlockSpec((NC, R, D + 16), lambda i: (0, i, 0)),
                  pl.BlockSpec((R, D), lambda i: (i, 0)),
                  pl.BlockSpec((1, D), lambda i: (0, 0)),
                  pl.BlockSpec((D, M), lambda i: (0, 0))],
        out_specs=pl.BlockSpec((R, M), lambda i: (i, 0)),
        out_shape=jax.ShapeDtypeStruct((N, M), jnp.float32),
    )(acc, skip, bias, w)


def _final(acc, skip, bias):
    R = 1000

    def body(a_ref, s_ref, b_ref, o_ref):
        raw = a_ref[0] + a_ref[1]
        ns = raw[:, :16]
        den = raw[:, 16:17]
        o_ref[...] = ns / (den + 1e-16) + s_ref[...] + b_ref[...]

    return pl.pallas_call(
        body,
        grid=(N // R,),
        in_specs=[pl.BlockSpec((NC, R, 32), lambda i: (0, i, 0)),
                  pl.BlockSpec((R, 16), lambda i: (i, 0)),
                  pl.BlockSpec((1, 16), lambda i: (0, 0))],
        out_specs=pl.BlockSpec((R, 16), lambda i: (i, 0)),
        out_shape=jax.ShapeDtypeStruct((N, 16), jnp.float32),
    )(acc, skip, bias)


def kernel(x, edge_index, Wl1, Wr1, att1, b1, Wlin1, blin1,
           Wl2, Wr2, att2, b2, Wlin2, blin2,
           Wlo, Wro, atto, bo, Wlino, blino):
    src = edge_index[0]
    dst = edge_index[1]

    zer144 = jnp.zeros((RPT, 144), jnp.float32)
    X1 = _matmul(x, jnp.concatenate([Wl1, Wr1, Wlin1], axis=1))
    acc1 = _edge_phase(128)(X1[:, :128], X1[:, 128:256], src, dst, att1,
                            zer144)

    X2 = _combine_matmul(acc1, X1[:, 256:], (b1 + blin1)[None, :],
                         jnp.concatenate([Wl2, Wr2, Wlin2], axis=1))
    acc2 = _edge_phase(128)(X2[:, :128], X2[:, 128:256], src, dst, att2,
                            zer144)

    z = jnp.zeros((128, 15), jnp.float32)
    W3 = jnp.concatenate([Wlo, z, Wro, z, Wlino, z], axis=1)
    X3 = _combine_matmul(acc2, X2[:, 256:], (b2 + blin2)[None, :], W3)

    att3 = jnp.concatenate([atto, jnp.zeros((15,), jnp.float32)])
    acc3 = _edge_phase(16)(X3[:, :16], X3[:, 16:32], src, dst, att3,
                           jnp.zeros((RPT, 32), jnp.float32))

    bc = jnp.broadcast_to((bo + blino)[None, :], (1, 16))
    out16 = _final(acc3, X3[:, 32:48], bc)
    return out16[:, 0:1]
